# Initial kernel scaffold; baseline (speedup 1.0000x reference)
import jax
import jax.numpy as jnp
from jax.experimental import pallas as pl


def kernel(inputs):
    raise NotImplementedError("write your pallas kernel here")


# speedup vs baseline: 3.1284x; 3.1284x over previous
import jax
import jax.numpy as jnp
from jax import lax
from jax.experimental import pallas as pl
from jax.experimental.pallas import tpu as pltpu
from jax.experimental.pallas import tpu_sc as plsc

NC, NS, L = 2, 16, 16
NW = NC * NS
R, N = 128, 32768
ROWS_PER_W = R // NW
NVR = N // L
BLK = 8
NBLK = NVR // BLK
K = 50
OUTW = 72

NEG_INF = float("-inf")
BIG_I = 0x3FFFFFFF


def _lane():
    return lax.iota(jnp.int32, L)


def _shuffle(a, perm):
    return lax.gather(
        a, perm.reshape(L, 1),
        lax.GatherDimensionNumbers(
            offset_dims=(), collapsed_slice_dims=(0,), start_index_map=(0,)),
        (1,), mode=lax.GatherScatterMode.PROMISE_IN_BOUNDS)


def _bfly(a, op):
    lane = _lane()
    for sh in (1, 2, 4, 8):
        a = op(a, _shuffle(a, lane ^ sh))
    return a


def _scalar(a):
    return jnp.squeeze(lax.slice(a, (0,), (1,)))


def _vsqrt(x):
    bits = lax.bitcast_convert_type(x, jnp.int32)
    y = lax.bitcast_convert_type(
        (bits >> 1) + jnp.int32(0x1FBD1DF5), jnp.float32)
    for _ in range(3):
        y = 0.5 * (y + x / y)
    return y


def _sc_body(x_hbm, out_hbm, row_v, bufv, bufi, stats):
    wid = lax.axis_index("s") * NC + lax.axis_index("c")
    lane = _lane()
    ninf = jnp.full((L,), NEG_INF, jnp.float32)
    zero = jnp.zeros((L,), jnp.float32)

    def do_row(r, _):
        row = wid * ROWS_PER_W + r
        pltpu.sync_copy(x_hbm.at[row], row_v)

        def p1(i, ts):
            t0, t1, t2, t3 = ts
            for u in range(BLK):
                v = row_v[pl.ds((i * BLK + u) * L, L)]
                hi = jnp.maximum(t0, v); v = jnp.minimum(t0, v); t0 = hi
                hi = jnp.maximum(t1, v); v = jnp.minimum(t1, v); t1 = hi
                hi = jnp.maximum(t2, v); v = jnp.minimum(t2, v); t2 = hi
                t3 = jnp.maximum(t3, v)
            return t0, t1, t2, t3

        _, _, _, t3 = lax.fori_loop(0, NBLK, p1, (ninf, ninf, ninf, ninf))
        t = _scalar(_bfly(t3, jnp.minimum))

        def p2(b, ptr):
            base = b * (BLK * L)
            vs = [row_v[pl.ds(base + k * L, L)] for k in range(BLK)]
            bm = vs[0]
            for k in range(1, BLK):
                bm = jnp.maximum(bm, vs[k])
            hit = _scalar(_bfly(bm, jnp.maximum)) >= t

            def append(p):
                hm = jnp.zeros((L,), jnp.int32)
                for k in range(BLK):
                    hm = hm | jnp.where(vs[k] >= t, 1 << k, 0)
                hmsk = _scalar(_bfly(hm, lambda a, b: a | b))
                for k in range(BLK):
                    def app(q, k=k):
                        bufv[pl.ds(q, L)] = jnp.where(vs[k] >= t, vs[k],
                                                      NEG_INF)
                        bufi[pl.ds(q, L)] = lane + (base + k * L)
                        return q + L
                    p = lax.cond(((hmsk >> k) & 1) == 1, app,
                                 lambda q: q, p)
                return p

            return lax.cond(hit, append, lambda p: p, ptr)

        ptr = lax.fori_loop(0, NBLK, p2, jnp.int32(0))
        nb = ptr // L

        def emit(j, carry):
            sumsq, sumidx, maxabs, maxif, o0, o1, o2, o3 = carry

            def q1(k, acc):
                return jnp.maximum(acc, bufv[pl.ds(k * L, L)])

            m = _bfly(lax.fori_loop(0, nb, q1, ninf), jnp.maximum)

            def q2(k, acc):
                v = bufv[pl.ds(k * L, L)]
                ii = bufi[pl.ds(k * L, L)]
                return jnp.minimum(acc, jnp.where(v == m, ii, BIG_I))

            mi = _bfly(
                lax.fori_loop(0, nb, q2, jnp.full((L,), BIG_I, jnp.int32)),
                jnp.minimum)

            def q3(k, _):
                v = bufv[pl.ds(k * L, L)]
                ii = bufi[pl.ds(k * L, L)]
                bufv[pl.ds(k * L, L)] = jnp.where((v == m) & (ii == mi),
                                                  NEG_INF, v)
                return 0

            lax.fori_loop(0, nb, q3, 0)

            mif = mi.astype(jnp.float32)
            in10 = j < 10
            sumsq = sumsq + jnp.where(in10, m * m, zero)
            sumidx = sumidx + jnp.where(in10, mif, zero)
            isf = j == 0
            maxabs = jnp.where(isf, jnp.abs(m), maxabs)
            maxif = jnp.where(isf, mif, maxif)
            o0 = o0 + jnp.where(lane == j, mif, zero)
            o1 = o1 + jnp.where(lane == j - 16, mif, zero)
            o2 = o2 + jnp.where(lane == j - 32, mif, zero)
            o3 = o3 + jnp.where(lane == j - 48, mif, zero)
            return sumsq, sumidx, maxabs, maxif, o0, o1, o2, o3

        sumsq, sumidx, maxabs, maxif, o0, o1, o2, o3 = lax.fori_loop(
            0, K, emit, (zero, zero, zero, zero, zero, zero, zero, zero))

        rms10 = _vsqrt(sumsq * 0.1)
        s0 = (jnp.where(lane == 0, sumidx * 0.1, zero)
              + jnp.where(lane == 1, rms10, zero)
              + jnp.where(lane == 2, maxif, zero)
              + jnp.where(lane == 3, maxabs, zero))
        stats[pl.ds(0, L)] = s0
        stats[pl.ds(8, L)] = o0
        stats[pl.ds(24, L)] = o1
        stats[pl.ds(40, L)] = o2
        stats[pl.ds(56, L)] = o3
        pltpu.sync_copy(stats, out_hbm.at[row])
        return 0

    lax.fori_loop(0, ROWS_PER_W, do_row, 0)


@jax.jit
def kernel(inputs):
    mesh = plsc.VectorSubcoreMesh(core_axis_name="c", subcore_axis_name="s")
    padded = pl.kernel(
        _sc_body,
        out_type=jax.ShapeDtypeStruct((R, OUTW), jnp.float32),
        mesh=mesh,
        scratch_types=[
            pltpu.VMEM((N,), jnp.float32),
            pltpu.VMEM((N,), jnp.float32),
            pltpu.VMEM((N,), jnp.int32),
            pltpu.VMEM((OUTW,), jnp.float32),
        ],
    )(inputs)
    return jnp.concatenate([padded[:, 0:4], padded[:, 8:58]], axis=1)
